# baseline (speedup 1.0000x reference)
import functools

import jax
import jax.numpy as jnp
from jax import lax
from jax.experimental import pallas as pl
from jax.experimental.pallas import tpu as pltpu
from jax.experimental.pallas import tpu_sc as plsc

_NC = 2
_NS = 16
_NW = _NC * _NS
_L = 16


def _sc_body(x_hbm, t_hbm, o_hbm, t_v, x_v, *, B, S, D, CH, rows_per_w):
    wid = lax.axis_index("s") * _NC + lax.axis_index("c")
    base = wid * rows_per_w
    for c in range(rows_per_w // CH):
        row0 = base + c * CH
        pltpu.sync_copy(t_hbm.at[pl.ds(row0, CH)], t_v)
        for b in range(B):
            r = b * S + row0
            pltpu.sync_copy(x_hbm.at[pl.ds(r, CH)], x_v)

            @pl.loop(0, CH)
            def _row(i):
                @plsc.parallel_loop(0, D // _L, unroll=8)
                def _col(j):
                    plsc.addupdate(
                        x_v.at[i, pl.ds(j * _L, _L)],
                        t_v[i, pl.ds(j * _L, _L)],
                    )

            pltpu.sync_copy(x_v, o_hbm.at[pl.ds(r, CH)])


def kernel(inputs, pos_table):
    B, S, D = inputs.shape
    CH = 32
    rows_per_w = S // _NW
    mesh = plsc.VectorSubcoreMesh(core_axis_name="c", subcore_axis_name="s")

    body = functools.partial(_sc_body, B=B, S=S, D=D, CH=CH, rows_per_w=rows_per_w)
    k = pl.kernel(
        body,
        out_type=jax.ShapeDtypeStruct((B * S, D), inputs.dtype),
        mesh=mesh,
        scratch_types=[
            pltpu.VMEM((CH, D), inputs.dtype),
            pltpu.VMEM((CH, D), inputs.dtype),
        ],
    )
    out = k(inputs.reshape(B * S, D), pos_table)
    return out.reshape(B, S, D)


# speedup vs baseline: 1.4310x; 1.4310x over previous
import functools

import jax
import jax.numpy as jnp
from jax import lax
from jax.experimental import pallas as pl
from jax.experimental.pallas import tpu as pltpu
from jax.experimental.pallas import tpu_sc as plsc

_NC = 2
_NS = 16
_NW = _NC * _NS
_L = 16


def _sc_body(x_hbm, t_hbm, o_hbm, t_v, xv0, xv1,
             sem_t, sem_g0, sem_g1, sem_s0, sem_s1,
             *, Bk, S, D, CH, rows_per_w):
    wid = lax.axis_index("s") * _NC + lax.axis_index("c")
    base = wid * rows_per_w
    chunks = rows_per_w // CH
    n_steps = chunks * Bk

    xbufs = [xv0, xv1]
    gsems = [sem_g0, sem_g1]
    ssems = [sem_s0, sem_s1]
    flat = CH * D

    def x_row(step):
        c, b = divmod(step, Bk)
        return b * S + base + c * CH

    def start_xgather(step, j):
        return pltpu.async_copy(x_hbm.at[pl.ds(x_row(step), CH)], xbufs[j], gsems[j])

    t_copy = pltpu.async_copy(t_hbm.at[pl.ds(base, CH)], t_v, sem_t)
    gathers = [start_xgather(0, 0), None]
    scatters = [None, None]

    for step in range(n_steps):
        j = step & 1
        c, b = divmod(step, Bk)
        if step + 1 < n_steps:
            if scatters[j ^ 1] is not None:
                scatters[j ^ 1].wait()
            gathers[j ^ 1] = start_xgather(step + 1, j ^ 1)
        gathers[j].wait()
        if b == 0:
            t_copy.wait()

        x_v = xbufs[j]

        @pl.loop(0, CH)
        def _row(r):
            @plsc.parallel_loop(0, D // _L, unroll=8)
            def _add(i):
                plsc.addupdate(x_v.at[r, pl.ds(i * _L, _L)],
                               t_v[r, pl.ds(i * _L, _L)])

        if b == Bk - 1 and c + 1 < chunks:
            t_copy = pltpu.async_copy(
                t_hbm.at[pl.ds(base + (c + 1) * CH, CH)], t_v, sem_t)
        scatters[j] = pltpu.async_copy(
            xbufs[j], o_hbm.at[pl.ds(x_row(step), CH)], ssems[j])

    for sc in scatters:
        if sc is not None:
            sc.wait()


def kernel(inputs, pos_table):
    B, S, D = inputs.shape
    CH = 32
    rows_per_w = S // _NW
    mesh = plsc.VectorSubcoreMesh(core_axis_name="c", subcore_axis_name="s")

    body = functools.partial(_sc_body, Bk=B, S=S, D=D, CH=CH,
                             rows_per_w=rows_per_w)
    k = pl.kernel(
        body,
        out_type=jax.ShapeDtypeStruct((B * S, D), inputs.dtype),
        mesh=mesh,
        scratch_types=[
            pltpu.VMEM((CH, D), inputs.dtype),
            pltpu.VMEM((CH, D), inputs.dtype),
            pltpu.VMEM((CH, D), inputs.dtype),
            pltpu.SemaphoreType.DMA,
            pltpu.SemaphoreType.DMA,
            pltpu.SemaphoreType.DMA,
            pltpu.SemaphoreType.DMA,
            pltpu.SemaphoreType.DMA,
        ],
    )
    out = k(inputs.reshape(B * S, D), pos_table)
    return out.reshape(B, S, D)
